# Initial kernel scaffold; baseline (speedup 1.0000x reference)
import jax
import jax.numpy as jnp
from jax.experimental import pallas as pl


def kernel(x, edge_index, emb_W, emb_b, gcn_W, gcn_b, bn1_g, bn1_b, attn_in_W, attn_in_b, attn_out_W, attn_out_b, bn2_g, bn2_b, mlp_W1, mlp_b1, mlp_W2, mlp_b2, bn3_g, bn3_b, cls_W, cls_b):
    raise NotImplementedError("write your pallas kernel here")


# speedup vs baseline: 1.5737x; 1.5737x over previous
import functools
from functools import partial

import jax
import jax.numpy as jnp
from jax import lax
from jax.experimental import pallas as pl
from jax.experimental.pallas import tpu as pltpu
from jax.experimental.pallas import tpu_sc as plsc

N = 2048
E = 8192
D = 128
L = 6
H = 4
DH = D // H
NCLS = 8

BQ = 256
NQ = N // BQ
BK = 256
KB = N // BK
BN_SCALE = 1.0 / (1.0 + 1e-5) ** 0.5

NW = 32
EPW = E // NW
ECH = 128
NJ = EPW // ECH
ROWS_PER_TILE = N // 16


def _sc_scatter_body(table_hbm, row_hbm, col_hbm, zeros_hbm, out_hbm,
                     idx_v, cidx_v, rows_v, obuf_v, acc_sh, sem):
    c = lax.axis_index("c")
    s = lax.axis_index("s")
    wid = s * 2 + c
    base = s * ROWS_PER_TILE
    pltpu.sync_copy(zeros_hbm.at[pl.ds(base, ROWS_PER_TILE)], obuf_v)
    pltpu.sync_copy(obuf_v, acc_sh.at[pl.ds(base, ROWS_PER_TILE)])
    plsc.subcore_barrier()
    for j in range(NJ):
        pltpu.sync_copy(row_hbm.at[wid, j], idx_v)
        pltpu.async_copy(table_hbm.at[idx_v], rows_v, sem).wait()
        pltpu.sync_copy(col_hbm.at[wid, j], cidx_v)
        pltpu.sync_copy(rows_v, acc_sh.at[cidx_v], add=True)
    plsc.subcore_barrier()
    pltpu.sync_copy(acc_sh.at[pl.ds(base, ROWS_PER_TILE)], obuf_v)
    pltpu.sync_copy(obuf_v, out_hbm.at[c, pl.ds(base, ROWS_PER_TILE)])


@jax.jit
def _sc_scatter(table, row3, col3, zeros_t):
    return pl.kernel(
        _sc_scatter_body,
        out_type=jax.ShapeDtypeStruct((2, N, D), jnp.float32),
        mesh=plsc.VectorSubcoreMesh(core_axis_name="c", subcore_axis_name="s"),
        scratch_types=[
            pltpu.VMEM((ECH,), jnp.int32),
            pltpu.VMEM((ECH,), jnp.int32),
            pltpu.VMEM((ECH, D), jnp.float32),
            pltpu.VMEM((ROWS_PER_TILE, D), jnp.float32),
            pltpu.VMEM_SHARED((N, D), jnp.float32),
            pltpu.SemaphoreType.DMA,
        ],
    )(table, row3, col3, zeros_t)


def _dotT(a, b):
    return lax.dot_general(a, b, (((1,), (1,)), ((), ())),
                           preferred_element_type=jnp.float32)


def _dot(a, b):
    return lax.dot_general(a, b, (((1,), (0,)), ((), ())),
                           preferred_element_type=jnp.float32)


def _prologue_body(x_ref, embW_ref, embb_ref, W0_ref, degS_ref,
                   h0_ref, dinv_ref, g0_ref):
    deg = degS_ref[0] + degS_ref[1] + 1.0
    dinv = lax.rsqrt(deg)
    xe = _dotT(x_ref[...], embW_ref[...]) + embb_ref[...]
    h0 = jnp.where(xe > 0, xe, 0.01 * xe)
    h0_ref[...] = h0
    dinv_ref[...] = dinv
    g0_ref[...] = dinv * _dotT(h0, W0_ref[...])


def _prologue(x, embW, embb2, W0, degS):
    return pl.pallas_call(
        _prologue_body,
        grid=(NQ,),
        in_specs=[
            pl.BlockSpec((BQ, 2), lambda i: (i, 0)),
            pl.BlockSpec((D, 2), lambda i: (0, 0)),
            pl.BlockSpec((1, D), lambda i: (0, 0)),
            pl.BlockSpec((D, D), lambda i: (0, 0)),
            pl.BlockSpec((2, BQ, D), lambda i: (0, i, 0)),
        ],
        out_specs=[
            pl.BlockSpec((BQ, D), lambda i: (i, 0)),
            pl.BlockSpec((BQ, D), lambda i: (i, 0)),
            pl.BlockSpec((BQ, D), lambda i: (i, 0)),
        ],
        out_shape=[
            jax.ShapeDtypeStruct((N, D), jnp.float32),
            jax.ShapeDtypeStruct((N, D), jnp.float32),
            jax.ShapeDtypeStruct((N, D), jnp.float32),
        ],
    )(x, embW, embb2, W0, degS)


def _pre_body(h_ref, S_ref, g_ref, dinv_ref, gcnb_ref, bn1g_ref, bn1b_ref,
              inW_ref, bq_ref, bk_ref, bv_ref,
              h1_ref, q_ref, kt_ref, v_ref):
    h = h_ref[...]
    agg = dinv_ref[...] * (S_ref[0] + S_ref[1] + g_ref[...]) + gcnb_ref[...]
    h1_ref[...] = (agg + h) * (BN_SCALE * bn1g_ref[...]) + bn1b_ref[...]
    Wq = inW_ref[0:D, :]
    Wk = inW_ref[D:2 * D, :]
    Wv = inW_ref[2 * D:3 * D, :]
    q_ref[...] = _dotT(h, Wq) + bq_ref[...]
    kt_ref[...] = _dotT(Wk, h) + bk_ref[...]
    v_ref[...] = _dotT(h, Wv) + bv_ref[...]


def _pre(h, S, g, dinv, gcnb2, bn1g2, bn1b2, inW, bq2, bkc, bv2):
    return pl.pallas_call(
        _pre_body,
        grid=(NQ,),
        in_specs=[
            pl.BlockSpec((BQ, D), lambda i: (i, 0)),
            pl.BlockSpec((2, BQ, D), lambda i: (0, i, 0)),
            pl.BlockSpec((BQ, D), lambda i: (i, 0)),
            pl.BlockSpec((BQ, D), lambda i: (i, 0)),
            pl.BlockSpec((1, D), lambda i: (0, 0)),
            pl.BlockSpec((1, D), lambda i: (0, 0)),
            pl.BlockSpec((1, D), lambda i: (0, 0)),
            pl.BlockSpec((3 * D, D), lambda i: (0, 0)),
            pl.BlockSpec((1, D), lambda i: (0, 0)),
            pl.BlockSpec((D, 1), lambda i: (0, 0)),
            pl.BlockSpec((1, D), lambda i: (0, 0)),
        ],
        out_specs=[
            pl.BlockSpec((BQ, D), lambda i: (i, 0)),
            pl.BlockSpec((BQ, D), lambda i: (i, 0)),
            pl.BlockSpec((D, BQ), lambda i: (0, i)),
            pl.BlockSpec((BQ, D), lambda i: (i, 0)),
        ],
        out_shape=[
            jax.ShapeDtypeStruct((N, D), jnp.float32),
            jax.ShapeDtypeStruct((N, D), jnp.float32),
            jax.ShapeDtypeStruct((D, N), jnp.float32),
            jax.ShapeDtypeStruct((N, D), jnp.float32),
        ],
    )(h, S, g, dinv, gcnb2, bn1g2, bn1b2, inW, bq2, bkc, bv2)


def _attn_body(q_ref, kt_ref, v_ref, o_ref, s_ref):
    scale = 1.0 / (float(DH) ** 0.5)
    for hh in range(H):
        qh = q_ref[:, hh * DH:(hh + 1) * DH]
        m = jnp.full((BQ, 1), -jnp.inf, jnp.float32)
        for kb in range(KB):
            sc = _dot(qh, kt_ref[hh * DH:(hh + 1) * DH,
                                 kb * BK:(kb + 1) * BK]) * scale
            s_ref[:, kb * BK:(kb + 1) * BK] = sc
            m = jnp.maximum(m, jnp.max(sc, axis=1, keepdims=True))
        acc = jnp.zeros((BQ, DH), jnp.float32)
        den = jnp.zeros((BQ, 1), jnp.float32)
        for kb in range(KB):
            e = jnp.exp(s_ref[:, kb * BK:(kb + 1) * BK] - m)
            den = den + jnp.sum(e, axis=1, keepdims=True)
            acc = acc + _dot(e, v_ref[kb * BK:(kb + 1) * BK,
                                      hh * DH:(hh + 1) * DH])
        o_ref[:, hh * DH:(hh + 1) * DH] = acc / den


def _attn(q, kt, v):
    return pl.pallas_call(
        _attn_body,
        grid=(NQ,),
        in_specs=[
            pl.BlockSpec((BQ, D), lambda i: (i, 0)),
            pl.BlockSpec((D, N), lambda i: (0, 0)),
            pl.BlockSpec((N, D), lambda i: (0, 0)),
        ],
        out_specs=pl.BlockSpec((BQ, D), lambda i: (i, 0)),
        out_shape=jax.ShapeDtypeStruct((N, D), jnp.float32),
        scratch_shapes=[pltpu.VMEM((BQ, N), jnp.float32)],
    )(q, kt, v)


def _post_body(o_ref, h_ref, h1_ref, outW_ref, outb_ref, bn2g_ref, bn2b_ref,
               W1_ref, b1_ref, W2_ref, b2_ref, bn3g_ref, bn3b_ref,
               Wn_ref, dinv_ref, hn_ref, gn_ref):
    h = h_ref[...]
    h2 = _dotT(o_ref[...], outW_ref[...]) + outb_ref[...]
    h2 = (h2 + h) * (BN_SCALE * bn2g_ref[...]) + bn2b_ref[...]
    out = h1_ref[...] + h2
    mm = _dotT(out, W1_ref[...]) + b1_ref[...]
    mm = jnp.maximum(mm, 0.0)
    mm2 = _dotT(mm, W2_ref[...]) + b2_ref[...]
    out2 = (out + mm2) * (BN_SCALE * bn3g_ref[...]) + bn3b_ref[...]
    hn_ref[...] = out2
    gn_ref[...] = dinv_ref[...] * _dotT(out2, Wn_ref[...])


def _post(o, h, h1, outW, outb2, bn2g2, bn2b2, W1, b12, W2, b22,
          bn3g2, bn3b2, Wn, dinv):
    return pl.pallas_call(
        _post_body,
        grid=(NQ,),
        in_specs=[
            pl.BlockSpec((BQ, D), lambda i: (i, 0)),
            pl.BlockSpec((BQ, D), lambda i: (i, 0)),
            pl.BlockSpec((BQ, D), lambda i: (i, 0)),
            pl.BlockSpec((D, D), lambda i: (0, 0)),
            pl.BlockSpec((1, D), lambda i: (0, 0)),
            pl.BlockSpec((1, D), lambda i: (0, 0)),
            pl.BlockSpec((1, D), lambda i: (0, 0)),
            pl.BlockSpec((2 * D, D), lambda i: (0, 0)),
            pl.BlockSpec((1, 2 * D), lambda i: (0, 0)),
            pl.BlockSpec((D, 2 * D), lambda i: (0, 0)),
            pl.BlockSpec((1, D), lambda i: (0, 0)),
            pl.BlockSpec((1, D), lambda i: (0, 0)),
            pl.BlockSpec((1, D), lambda i: (0, 0)),
            pl.BlockSpec((D, D), lambda i: (0, 0)),
            pl.BlockSpec((BQ, D), lambda i: (i, 0)),
        ],
        out_specs=[
            pl.BlockSpec((BQ, D), lambda i: (i, 0)),
            pl.BlockSpec((BQ, D), lambda i: (i, 0)),
        ],
        out_shape=[
            jax.ShapeDtypeStruct((N, D), jnp.float32),
            jax.ShapeDtypeStruct((N, D), jnp.float32),
        ],
    )(o, h, h1, outW, outb2, bn2g2, bn2b2, W1, b12, W2, b22,
      bn3g2, bn3b2, Wn, dinv)


def _epilogue_body(h_ref, clsW_ref, clsb_ref, out_ref, acc_ref):
    i = pl.program_id(0)

    @pl.when(i == 0)
    def _():
        acc_ref[...] = jnp.zeros((1, D), jnp.float32)

    acc_ref[...] += jnp.sum(h_ref[...], axis=0, keepdims=True)

    @pl.when(i == NQ - 1)
    def _():
        pooled = acc_ref[...] * (1.0 / N)
        out_ref[...] = _dotT(pooled, clsW_ref[...]) + clsb_ref[...]


def _epilogue(h, clsW, clsb2):
    return pl.pallas_call(
        _epilogue_body,
        grid=(NQ,),
        in_specs=[
            pl.BlockSpec((BQ, D), lambda i: (i, 0)),
            pl.BlockSpec((NCLS, D), lambda i: (0, 0)),
            pl.BlockSpec((1, NCLS), lambda i: (0, 0)),
        ],
        out_specs=pl.BlockSpec((1, NCLS), lambda i: (0, 0)),
        out_shape=jax.ShapeDtypeStruct((1, NCLS), jnp.float32),
        scratch_shapes=[pltpu.VMEM((1, D), jnp.float32)],
    )(h, clsW, clsb2)


def kernel(x, edge_index, emb_W, emb_b, gcn_W, gcn_b, bn1_g, bn1_b,
           attn_in_W, attn_in_b, attn_out_W, attn_out_b, bn2_g, bn2_b,
           mlp_W1, mlp_b1, mlp_W2, mlp_b2, bn3_g, bn3_b, cls_W, cls_b):
    row3 = edge_index[0].reshape(NW, NJ, ECH)
    col3 = edge_index[1].reshape(NW, NJ, ECH)
    zeros_t = jnp.zeros((N, D), jnp.float32)
    ones_t = jnp.ones((N, D), jnp.float32)

    degS = _sc_scatter(ones_t, row3, col3, zeros_t)
    h, dinv, g = _prologue(x, emb_W, emb_b.reshape(1, D), gcn_W[0], degS)

    for i in range(L):
        S = _sc_scatter(g, row3, col3, zeros_t)
        inb = attn_in_b[i]
        h1, q, kt, v = _pre(
            h, S, g, dinv, gcn_b[i].reshape(1, D),
            bn1_g[i].reshape(1, D), bn1_b[i].reshape(1, D),
            attn_in_W[i], inb[0:D].reshape(1, D),
            inb[D:2 * D].reshape(D, 1), inb[2 * D:3 * D].reshape(1, D))
        o = _attn(q, kt, v)
        h, g = _post(
            o, h, h1, attn_out_W[i], attn_out_b[i].reshape(1, D),
            bn2_g[i].reshape(1, D), bn2_b[i].reshape(1, D),
            mlp_W1[i], mlp_b1[i].reshape(1, 2 * D),
            mlp_W2[i], mlp_b2[i].reshape(1, D),
            bn3_g[i].reshape(1, D), bn3_b[i].reshape(1, D),
            gcn_W[(i + 1) % L], dinv)

    logits = _epilogue(h, cls_W, cls_b.reshape(1, NCLS))
    return logits.reshape(NCLS)
